# Initial kernel scaffold; baseline (speedup 1.0000x reference)
import functools

import jax
import jax.numpy as jnp
from jax import lax
from jax.experimental import pallas as pl
from jax.experimental.pallas import tpu as pltpu
from jax.experimental.pallas import tpu_sc as plsc

N = 10000
E = 320000
D = 128
DW = 16
NC, NS = 2, 16
NW = NC * NS
B = 80
NB = E // (NW * B)
RPT = N // NS
RCH = 125

_mesh = plsc.VectorSubcoreMesh(
    core_axis_name="c", subcore_axis_name="s", num_cores=NC, num_subcores=NS)


@functools.partial(
    pl.kernel,
    out_type=jax.ShapeDtypeStruct((NC, N, DW), jnp.float32),
    mesh=_mesh,
    scratch_types=[
        pltpu.VMEM_SHARED((N, DW), jnp.float32),
        pltpu.VMEM((NB, B), jnp.int32),
        pltpu.VMEM((B, DW), jnp.float32),
        pltpu.VMEM((RPT, DW), jnp.float32),
    ],
)
def _deg_kernel(dst2d, ones_hbm, zeros_hbm, out, deg_sh, didx, ones_v, stage):
    c = lax.axis_index("c")
    s = lax.axis_index("s")
    w = c * NS + s
    pltpu.sync_copy(dst2d.at[pl.ds(w * NB, NB)], didx)
    pltpu.sync_copy(ones_hbm, ones_v)
    pltpu.sync_copy(zeros_hbm.at[pl.ds(s * RPT, RPT)], stage)
    pltpu.sync_copy(stage, deg_sh.at[pl.ds(s * RPT, RPT)])
    plsc.subcore_barrier()

    def body(j, carry):
        pltpu.sync_copy(ones_v, deg_sh.at[didx.at[j]], add=True)
        return carry

    lax.fori_loop(0, NB, body, None)
    plsc.subcore_barrier()
    pltpu.sync_copy(deg_sh.at[pl.ds(s * RPT, RPT)], stage)
    pltpu.sync_copy(stage, out.at[c, pl.ds(s * RPT, RPT)])


@functools.partial(
    pl.kernel,
    out_type=jax.ShapeDtypeStruct((NC, N, D), jnp.float32),
    mesh=_mesh,
    scratch_types=[
        pltpu.VMEM_SHARED((N, D), jnp.float32),
        pltpu.VMEM((NB, B), jnp.int32),
        pltpu.VMEM((NB, B), jnp.int32),
        pltpu.VMEM((B, D), jnp.float32),
        pltpu.VMEM((RCH, D), jnp.float32),
    ],
)
def _agg_kernel(src2d, dst2d, g_hbm, zeros_hbm, out, acc_sh, sidx, didx, rows,
                stage):
    c = lax.axis_index("c")
    s = lax.axis_index("s")
    w = c * NS + s
    pltpu.sync_copy(src2d.at[pl.ds(w * NB, NB)], sidx)
    pltpu.sync_copy(dst2d.at[pl.ds(w * NB, NB)], didx)
    for k in range(RPT // RCH):
        r0 = s * RPT + k * RCH
        pltpu.sync_copy(zeros_hbm.at[pl.ds(r0, RCH)], stage)
        pltpu.sync_copy(stage, acc_sh.at[pl.ds(r0, RCH)])
    plsc.subcore_barrier()

    def body(j, carry):
        pltpu.sync_copy(g_hbm.at[sidx.at[j]], rows)
        pltpu.sync_copy(rows, acc_sh.at[didx.at[j]], add=True)
        return carry

    lax.fori_loop(0, NB, body, None)
    plsc.subcore_barrier()
    for k in range(RPT // RCH):
        r0 = s * RPT + k * RCH
        pltpu.sync_copy(acc_sh.at[pl.ds(r0, RCH)], stage)
        pltpu.sync_copy(stage, out.at[c, pl.ds(r0, RCH)])


_RB = 1000


def _scale_body(x_ref, dga_ref, dgb_ref, g_ref):
    deg = dga_ref[:, 0:1] + dgb_ref[:, 0:1] + 1.0
    g_ref[...] = x_ref[...] * lax.rsqrt(deg)


def _tc_scale(x, dga, dgb):
    return pl.pallas_call(
        _scale_body,
        grid=(N // _RB,),
        in_specs=[
            pl.BlockSpec((_RB, D), lambda i: (i, 0)),
            pl.BlockSpec((_RB, DW), lambda i: (i, 0)),
            pl.BlockSpec((_RB, DW), lambda i: (i, 0)),
        ],
        out_specs=pl.BlockSpec((_RB, D), lambda i: (i, 0)),
        out_shape=jax.ShapeDtypeStruct((N, D), jnp.float32),
    )(x, dga, dgb)


def _final_body(acca_ref, accb_ref, g_ref, dga_ref, dgb_ref, w_ref, b_ref,
                o_ref):
    deg = dga_ref[:, 0:1] + dgb_ref[:, 0:1] + 1.0
    t = (acca_ref[...] + accb_ref[...] + g_ref[...]) * lax.rsqrt(deg)
    h = jnp.dot(t, w_ref[...], preferred_element_type=jnp.float32)
    h = h + b_ref[...]
    val = h[:, :D]
    gate = h[:, D:]
    o_ref[...] = val * (0.5 * gate * (1.0 + lax.erf(gate * 0.7071067811865476)))


def _tc_final(acca, accb, g, dga, dgb, W, b2):
    return pl.pallas_call(
        _final_body,
        grid=(N // _RB,),
        in_specs=[
            pl.BlockSpec((_RB, D), lambda i: (i, 0)),
            pl.BlockSpec((_RB, D), lambda i: (i, 0)),
            pl.BlockSpec((_RB, D), lambda i: (i, 0)),
            pl.BlockSpec((_RB, DW), lambda i: (i, 0)),
            pl.BlockSpec((_RB, DW), lambda i: (i, 0)),
            pl.BlockSpec((D, 2 * D), lambda i: (0, 0)),
            pl.BlockSpec((1, 2 * D), lambda i: (0, 0)),
        ],
        out_specs=pl.BlockSpec((_RB, D), lambda i: (i, 0)),
        out_shape=jax.ShapeDtypeStruct((N, D), jnp.float32),
    )(acca, accb, g, dga, dgb, W, b2)


def kernel(x, edge_index, W, b):
    src2d = edge_index[0].astype(jnp.int32).reshape(E // B, B)
    dst2d = edge_index[1].astype(jnp.int32).reshape(E // B, B)
    ones16 = jnp.ones((B, DW), jnp.float32)
    zdeg = jnp.zeros((N, DW), jnp.float32)
    zacc = jnp.zeros((N, D), jnp.float32)

    degp = _deg_kernel(dst2d, ones16, zdeg)
    g = _tc_scale(x, degp[0], degp[1])
    accp = _agg_kernel(src2d, dst2d, g, zacc)
    return _tc_final(accp[0], accp[1], g, degp[0], degp[1], W,
                     b.reshape(1, 2 * D))


# speedup vs baseline: 19.8291x; 19.8291x over previous
import functools

import jax
import jax.numpy as jnp
from jax import lax
from jax.experimental import pallas as pl
from jax.experimental.pallas import tpu as pltpu
from jax.experimental.pallas import tpu_sc as plsc

N = 10000
E = 320000
D = 128
DW = 16
NC, NS = 2, 16
NW = NC * NS
B = 80
EPW = E // NW
NB = EPW // B
RPS = 640
NP = NS * RPS

_mesh = plsc.VectorSubcoreMesh(
    core_axis_name="c", subcore_axis_name="s", num_cores=NC, num_subcores=NS)


@functools.partial(
    pl.kernel,
    out_type=jax.ShapeDtypeStruct((NC, NP), jnp.float32),
    mesh=_mesh,
    scratch_types=[
        pltpu.VMEM_SHARED((NP,), jnp.float32),
        pltpu.VMEM((B,), jnp.int32),
        pltpu.VMEM((B,), jnp.float32),
    ],
)
def _deg_kernel(dst1d, ones_hbm, zeros_hbm, out, deg_sh, didx, ones_v):
    c = lax.axis_index("c")
    s = lax.axis_index("s")
    w = c * NS + s
    pltpu.sync_copy(ones_hbm, ones_v)
    pltpu.sync_copy(zeros_hbm.at[pl.ds(s * RPS, RPS)],
                    deg_sh.at[pl.ds(s * RPS, RPS)])
    plsc.subcore_barrier()

    def body(j, carry):
        pltpu.sync_copy(dst1d.at[pl.ds(w * EPW + j * B, B)], didx)
        pltpu.sync_copy(ones_v, deg_sh.at[didx], add=True)
        return carry

    lax.fori_loop(0, NB, body, None)
    plsc.subcore_barrier()
    pltpu.sync_copy(deg_sh.at[pl.ds(s * RPS, RPS)],
                    out.at[c, pl.ds(s * RPS, RPS)])


@functools.partial(
    pl.kernel,
    out_type=jax.ShapeDtypeStruct((NC, NP, D), jnp.float32),
    mesh=_mesh,
    scratch_types=[
        pltpu.VMEM_SHARED((NP, D), jnp.float32),
        pltpu.VMEM((B,), jnp.int32),
        pltpu.VMEM((B,), jnp.int32),
        pltpu.VMEM((B, D), jnp.float32),
    ],
)
def _agg_kernel(src1d, dst1d, g_hbm, zeros_hbm, out, acc_sh, sidx, didx, rows):
    c = lax.axis_index("c")
    s = lax.axis_index("s")
    w = c * NS + s
    pltpu.sync_copy(zeros_hbm.at[pl.ds(s * RPS, RPS)],
                    acc_sh.at[pl.ds(s * RPS, RPS)])
    plsc.subcore_barrier()

    def body(j, carry):
        e0 = w * EPW + j * B
        pltpu.sync_copy(src1d.at[pl.ds(e0, B)], sidx)
        pltpu.sync_copy(dst1d.at[pl.ds(e0, B)], didx)
        pltpu.sync_copy(g_hbm.at[sidx], rows)
        pltpu.sync_copy(rows, acc_sh.at[didx], add=True)
        return carry

    lax.fori_loop(0, NB, body, None)
    plsc.subcore_barrier()
    pltpu.sync_copy(acc_sh.at[pl.ds(s * RPS, RPS)],
                    out.at[c, pl.ds(s * RPS, RPS)])


_RB = 1000


def _scale_body(x_ref, dga_ref, dgb_ref, g_ref):
    deg = dga_ref[...] + dgb_ref[...] + 1.0
    g_ref[...] = x_ref[...] * lax.rsqrt(deg)


def _tc_scale(x, dga, dgb):
    return pl.pallas_call(
        _scale_body,
        grid=(N // _RB,),
        in_specs=[
            pl.BlockSpec((_RB, D), lambda i: (i, 0)),
            pl.BlockSpec((_RB, 1), lambda i: (i, 0)),
            pl.BlockSpec((_RB, 1), lambda i: (i, 0)),
        ],
        out_specs=pl.BlockSpec((_RB, D), lambda i: (i, 0)),
        out_shape=jax.ShapeDtypeStruct((N, D), jnp.float32),
    )(x, dga, dgb)


def _final_body(acca_ref, accb_ref, g_ref, dga_ref, dgb_ref, w_ref, b_ref,
                o_ref):
    deg = dga_ref[...] + dgb_ref[...] + 1.0
    t = (acca_ref[...] + accb_ref[...] + g_ref[...]) * lax.rsqrt(deg)
    h = jnp.dot(t, w_ref[...], preferred_element_type=jnp.float32)
    h = h + b_ref[...]
    val = h[:, :D]
    gate = h[:, D:]
    o_ref[...] = val * (0.5 * gate * (1.0 + lax.erf(gate * 0.7071067811865476)))


def _tc_final(acca, accb, g, dga, dgb, W, b2):
    return pl.pallas_call(
        _final_body,
        grid=(N // _RB,),
        in_specs=[
            pl.BlockSpec((_RB, D), lambda i: (i, 0)),
            pl.BlockSpec((_RB, D), lambda i: (i, 0)),
            pl.BlockSpec((_RB, D), lambda i: (i, 0)),
            pl.BlockSpec((_RB, 1), lambda i: (i, 0)),
            pl.BlockSpec((_RB, 1), lambda i: (i, 0)),
            pl.BlockSpec((D, 2 * D), lambda i: (0, 0)),
            pl.BlockSpec((1, 2 * D), lambda i: (0, 0)),
        ],
        out_specs=pl.BlockSpec((_RB, D), lambda i: (i, 0)),
        out_shape=jax.ShapeDtypeStruct((N, D), jnp.float32),
    )(acca, accb, g, dga, dgb, W, b2)


def kernel(x, edge_index, W, b):
    src1d = edge_index[0].astype(jnp.int32)
    dst1d = edge_index[1].astype(jnp.int32)
    ones1 = jnp.ones((B,), jnp.float32)
    zdeg = jnp.zeros((NP,), jnp.float32)
    zacc = jnp.zeros((NP, D), jnp.float32)

    degp = _deg_kernel(dst1d, ones1, zdeg)
    dga = degp[0].reshape(NP, 1)
    dgb = degp[1].reshape(NP, 1)
    g = _tc_scale(x, dga, dgb)
    accp = _agg_kernel(src1d, dst1d, g, zacc)
    return _tc_final(accp[0], accp[1], g, dga, dgb, W,
                     b.reshape(1, 2 * D))
